# Initial kernel scaffold; baseline (speedup 1.0000x reference)
import jax
import jax.numpy as jnp
from jax.experimental import pallas as pl


def kernel(idxs, emb):
    raise NotImplementedError("write your pallas kernel here")


# speedup vs baseline: 1.9386x; 1.9386x over previous
import functools

import jax
import jax.numpy as jnp
from jax import lax
from jax.experimental import pallas as pl
from jax.experimental.pallas import tpu as pltpu
from jax.experimental.pallas import tpu_sc as plsc

_DIM = 32
_NW = 32
_CHUNK = 512
_GSUB = 128


def _poe_pallas(idx1, idx2, emb):
    n = idx1.shape[0]
    per_w = n // _NW
    n_chunks = per_w // _CHUNK
    groups = _CHUNK // 16
    n_gather = _CHUNK // _GSUB

    mesh = plsc.VectorSubcoreMesh(
        core_axis_name="c", subcore_axis_name="s", num_cores=2, num_subcores=16
    )

    @functools.partial(
        pl.kernel,
        out_type=jax.ShapeDtypeStruct((n,), jnp.float32),
        mesh=mesh,
        compiler_params=pltpu.CompilerParams(
            needs_layout_passes=False, use_tc_tiling_on_sc=False),
        scratch_types=[
            pltpu.VMEM((_CHUNK,), jnp.int32),
            pltpu.VMEM((_CHUNK,), jnp.int32),
            pltpu.VMEM((_CHUNK, _DIM), jnp.float32),
            pltpu.VMEM((_CHUNK, _DIM), jnp.float32),
            pltpu.VMEM((_CHUNK,), jnp.float32),
            pltpu.SemaphoreType.DMA,
        ],
    )
    def run(idx1_hbm, idx2_hbm, emb_hbm, out_hbm,
            idx1_v, idx2_v, rows1_v, rows2_v, out_v, sem):
        wid = lax.axis_index("s") * 2 + lax.axis_index("c")
        base = wid * per_w
        lanes = lax.iota(jnp.int32, 16)

        def chunk_body(g, carry):
            start = base + g * _CHUNK
            pltpu.sync_copy(idx1_hbm.at[pl.ds(start, _CHUNK)], idx1_v)
            pltpu.sync_copy(idx2_hbm.at[pl.ds(start, _CHUNK)], idx2_v)
            copies = []
            for j in range(n_gather):
                sl = pl.ds(j * _GSUB, _GSUB)
                copies.append(
                    pltpu.async_copy(emb_hbm.at[idx1_v.at[sl]], rows1_v.at[sl], sem))
                copies.append(
                    pltpu.async_copy(emb_hbm.at[idx2_v.at[sl]], rows2_v.at[sl], sem))
            for cp in copies:
                cp.wait()

            def group_body(gi, c2):
                rows = gi * 16 + lanes
                acc = jnp.zeros((16,), jnp.float32)
                for d in range(_DIM):
                    col = jnp.full((16,), d, jnp.int32)
                    v1 = plsc.load_gather(rows1_v, [rows, col])
                    v2 = plsc.load_gather(rows2_v, [rows, col])
                    acc = acc + jnp.maximum(v1 - v2, 0.0)
                out_v[pl.ds(gi * 16, 16)] = -acc
                return c2

            lax.fori_loop(0, groups, group_body, 0)
            pltpu.sync_copy(out_v, out_hbm.at[pl.ds(start, _CHUNK)])
            return carry

        lax.fori_loop(0, n_chunks, chunk_body, 0)

    return run(idx1, idx2, emb)


def kernel(idxs, emb):
    b, s, _ = idxs.shape
    flat = idxs.reshape(-1, 2)
    out = _poe_pallas(flat[:, 0], flat[:, 1], emb)
    return out.reshape(b, s)


# speedup vs baseline: 1.9597x; 1.0109x over previous
import functools

import jax
import jax.numpy as jnp
from jax import lax
from jax.experimental import pallas as pl
from jax.experimental.pallas import tpu as pltpu
from jax.experimental.pallas import tpu_sc as plsc

_DIM = 32
_NW = 32
_CHUNK = 256
_GSUB = 128
_NSUB = _CHUNK // _GSUB


def _poe_pallas(idx1, idx2, emb):
    n = idx1.shape[0]
    per_w = n // _NW
    n_chunks = per_w // _CHUNK
    groups = _CHUNK // 16

    mesh = plsc.VectorSubcoreMesh(
        core_axis_name="c", subcore_axis_name="s", num_cores=2, num_subcores=16
    )

    @functools.partial(
        pl.kernel,
        out_type=jax.ShapeDtypeStruct((n,), jnp.float32),
        mesh=mesh,
        compiler_params=pltpu.CompilerParams(
            needs_layout_passes=False, use_tc_tiling_on_sc=False),
        scratch_types=[
            pltpu.VMEM((per_w,), jnp.int32),
            pltpu.VMEM((per_w,), jnp.int32),
            pltpu.VMEM((per_w,), jnp.float32),
            pltpu.VMEM((_CHUNK, _DIM), jnp.float32),
            pltpu.VMEM((_CHUNK, _DIM), jnp.float32),
            pltpu.VMEM((_CHUNK, _DIM), jnp.float32),
            pltpu.VMEM((_CHUNK, _DIM), jnp.float32),
            pltpu.SemaphoreType.DMA,
            pltpu.SemaphoreType.DMA,
        ],
    )
    def run(idx1_hbm, idx2_hbm, emb_hbm, out_hbm,
            idx1_v, idx2_v, out_v, r1a, r2a, r1b, r2b, sem_a, sem_b):
        wid = lax.axis_index("s") * 2 + lax.axis_index("c")
        base = wid * per_w
        lanes = lax.iota(jnp.int32, 16)

        pltpu.sync_copy(idx1_hbm.at[pl.ds(base, per_w)], idx1_v)
        pltpu.sync_copy(idx2_hbm.at[pl.ds(base, per_w)], idx2_v)

        def fire(g, r1, r2, sem):
            for j in range(_NSUB):
                src = pl.ds(g * _CHUNK + j * _GSUB, _GSUB)
                dst = pl.ds(j * _GSUB, _GSUB)
                pltpu.async_copy(emb_hbm.at[idx1_v.at[src]], r1.at[dst], sem)
                pltpu.async_copy(emb_hbm.at[idx2_v.at[src]], r2.at[dst], sem)

        def drain(r1, r2, sem):
            for j in range(_NSUB):
                dst = pl.ds(j * _GSUB, _GSUB)
                pltpu.make_async_copy(
                    emb_hbm.at[idx1_v.at[pl.ds(0, _GSUB)]], r1.at[dst], sem).wait()
                pltpu.make_async_copy(
                    emb_hbm.at[idx2_v.at[pl.ds(0, _GSUB)]], r2.at[dst], sem).wait()

        def compute(g, r1, r2):
            def group_body(gi, c2):
                rows = gi * 16 + lanes
                acc = jnp.zeros((16,), jnp.float32)
                for d in range(_DIM):
                    col = jnp.full((16,), d, jnp.int32)
                    v1 = plsc.load_gather(r1, [rows, col])
                    v2 = plsc.load_gather(r2, [rows, col])
                    acc = acc + jnp.maximum(v1 - v2, 0.0)
                out_v[pl.ds(g * _CHUNK + gi * 16, 16)] = -acc
                return c2
            lax.fori_loop(0, groups, group_body, 0, unroll=True)

        fire(0, r1a, r2a, sem_a)

        def pair_body(i, carry):
            g0 = i * 2
            fire(g0 + 1, r1b, r2b, sem_b)
            drain(r1a, r2a, sem_a)
            compute(g0, r1a, r2a)

            @pl.when(g0 + 2 < n_chunks)
            def _():
                fire(g0 + 2, r1a, r2a, sem_a)

            drain(r1b, r2b, sem_b)
            compute(g0 + 1, r1b, r2b)
            return carry

        lax.fori_loop(0, n_chunks // 2, pair_body, 0)
        pltpu.sync_copy(out_v, out_hbm.at[pl.ds(base, per_w)])

    return run(idx1, idx2, emb)


def kernel(idxs, emb):
    b, s, _ = idxs.shape
    flat = idxs.reshape(-1, 2)
    out = _poe_pallas(flat[:, 0], flat[:, 1], emb)
    return out.reshape(b, s)
